# baseline (speedup 1.0000x reference)
import functools

import jax
import jax.numpy as jnp
from jax import lax
from jax.experimental import pallas as pl
from jax.experimental.pallas import tpu as pltpu
from jax.experimental.pallas import tpu_sc as plsc


def kernel(playlist_ids, song_ids, playlist_table, song_table):
    (B,) = playlist_ids.shape
    V, D = playlist_table.shape
    info = plsc.get_sparse_core_info()
    NC, NS, L = info.num_cores, info.num_subcores, info.num_lanes
    NW = NC * NS
    assert B % (8 * NW) == 0
    b_per_w = B // NW

    mesh = plsc.VectorSubcoreMesh(core_axis_name="c", subcore_axis_name="s")

    @functools.partial(
        pl.kernel,
        mesh=mesh,
        compiler_params=pltpu.CompilerParams(
            needs_layout_passes=False, use_tc_tiling_on_sc=False
        ),
        out_type=jax.ShapeDtypeStruct((B,), jnp.float32),
        scratch_types=[
            pltpu.VMEM((b_per_w,), jnp.int32),
            pltpu.VMEM((b_per_w,), jnp.int32),
            pltpu.VMEM((b_per_w, D), jnp.float32),
            pltpu.VMEM((b_per_w, D), jnp.float32),
            pltpu.VMEM((b_per_w,), jnp.float32),
            pltpu.SemaphoreType.DMA,
            pltpu.SemaphoreType.DMA,
        ],
    )
    def sc_kernel(pid_hbm, sid_hbm, ptab_hbm, stab_hbm, out_hbm,
                  pidx_v, sidx_v, prow_v, srow_v, outb_v, psem, ssem):
        wid = lax.axis_index("s") * NC + lax.axis_index("c")
        base = wid * b_per_w
        pltpu.sync_copy(pid_hbm.at[pl.ds(base, b_per_w)], pidx_v)
        pltpu.sync_copy(sid_hbm.at[pl.ds(base, b_per_w)], sidx_v)
        pcopy = pltpu.async_copy(ptab_hbm.at[pidx_v], prow_v, psem)
        scopy = pltpu.async_copy(stab_hbm.at[sidx_v], srow_v, ssem)
        pcopy.wait()
        scopy.wait()

        def group_body(g, carry):
            rows = lax.iota(jnp.int32, L) + g * L
            acc = jnp.zeros((L,), jnp.float32)
            for d in range(D):
                cols = jnp.full((L,), d, jnp.int32)
                pv = plsc.load_gather(prow_v, [rows, cols])
                sv = plsc.load_gather(srow_v, [rows, cols])
                acc = acc + pv * sv
            outb_v[pl.ds(g * L, L)] = acc
            return carry

        lax.fori_loop(0, b_per_w // L, group_body, 0)
        pltpu.sync_copy(outb_v, out_hbm.at[pl.ds(base, b_per_w)])

    return sc_kernel(playlist_ids, song_ids, playlist_table, song_table)


# speedup vs baseline: 3.7486x; 3.7486x over previous
import functools

import jax
import jax.numpy as jnp
from jax import lax
from jax.experimental import pallas as pl
from jax.experimental.pallas import tpu as pltpu
from jax.experimental.pallas import tpu_sc as plsc


def kernel(playlist_ids, song_ids, playlist_table, song_table):
    (B,) = playlist_ids.shape
    V, D = playlist_table.shape
    info = plsc.get_sparse_core_info()
    NC, NS, L = info.num_cores, info.num_subcores, info.num_lanes
    NW = NC * NS
    assert B % (8 * NW) == 0
    b_per_w = B // NW
    n_chunks = b_per_w // L

    mesh = plsc.VectorSubcoreMesh(core_axis_name="c", subcore_axis_name="s")

    @functools.partial(
        pl.kernel,
        mesh=mesh,
        compiler_params=pltpu.CompilerParams(needs_layout_passes=False),
        out_type=jax.ShapeDtypeStruct((B,), jnp.float32),
        scratch_types=[
            pltpu.SMEM((b_per_w,), jnp.int32),
            pltpu.SMEM((b_per_w,), jnp.int32),
            pltpu.VMEM((b_per_w,), jnp.int32),
            pltpu.VMEM((b_per_w,), jnp.int32),
            pltpu.VMEM((L, D, 128), jnp.float32),
            pltpu.VMEM((D, L), jnp.float32),
            pltpu.VMEM((b_per_w,), jnp.float32),
            pltpu.SemaphoreType.DMA,
        ],
    )
    def sc_kernel(pid_hbm, sid_hbm, ptab_hbm, stab_hbm, out_hbm,
                  pidx_s, sidx_s, pidx_v, sidx_v, blk_v, pcomp_v, outb_v, sem):
        wid = lax.axis_index("s") * NC + lax.axis_index("c")
        base = wid * b_per_w
        pltpu.sync_copy(pid_hbm.at[pl.ds(base, b_per_w)], pidx_v)
        pltpu.sync_copy(sid_hbm.at[pl.ds(base, b_per_w)], sidx_v)
        def stage_body(g, carry):
            pv = pidx_v[pl.ds(g * L, L)]
            sv = sidx_v[pl.ds(g * L, L)]
            for j in range(L):
                pidx_s[g * L + j] = pv[j]
                sidx_s[g * L + j] = sv[j]
            return carry

        lax.fori_loop(0, n_chunks, stage_body, 0)

        ex_lanes = lax.iota(jnp.int32, L)

        def fetch_blocks(idx_s, tab_hbm, c):
            for e in range(L):
                blk = idx_s[c * L + e] >> 7
                off = pl.multiple_of(blk * 128, 128)
                pltpu.async_copy(
                    tab_hbm.at[:, pl.ds(off, 128)], blk_v.at[e], sem
                )
            for e in range(L):
                pltpu.make_async_copy(
                    tab_hbm.at[:, pl.ds(0, 128)], blk_v.at[e], sem
                ).wait()

        def chunk_body(c, carry):
            lanes_p = pidx_v[pl.ds(c * L, L)] & 127
            fetch_blocks(pidx_s, ptab_hbm, c)
            for d in range(D):
                pcomp_v[d, :] = plsc.load_gather(
                    blk_v, [ex_lanes, jnp.full((L,), d, jnp.int32), lanes_p]
                )
            lanes_s = sidx_v[pl.ds(c * L, L)] & 127
            fetch_blocks(sidx_s, stab_hbm, c)
            acc = pcomp_v[0, :] * plsc.load_gather(
                blk_v, [ex_lanes, jnp.zeros((L,), jnp.int32), lanes_s]
            )
            for d in range(1, D):
                sv = plsc.load_gather(
                    blk_v, [ex_lanes, jnp.full((L,), d, jnp.int32), lanes_s]
                )
                acc = acc + pcomp_v[d, :] * sv
            outb_v[pl.ds(c * L, L)] = acc
            return carry

        lax.fori_loop(0, n_chunks, chunk_body, 0)
        pltpu.sync_copy(outb_v, out_hbm.at[pl.ds(base, b_per_w)])

    return sc_kernel(
        playlist_ids, song_ids, playlist_table.T, song_table.T
    )


# speedup vs baseline: 3.7584x; 1.0026x over previous
import functools

import jax
import jax.numpy as jnp
from jax import lax
from jax.experimental import pallas as pl
from jax.experimental.pallas import tpu as pltpu
from jax.experimental.pallas import tpu_sc as plsc


def kernel(playlist_ids, song_ids, playlist_table, song_table):
    (B,) = playlist_ids.shape
    V, D = playlist_table.shape
    info = plsc.get_sparse_core_info()
    NC, NS, L = info.num_cores, info.num_subcores, info.num_lanes
    NW = NC * NS
    assert B % (8 * NW) == 0
    b_per_w = B // NW
    E = 4
    n_sub = b_per_w // E

    mesh = plsc.VectorSubcoreMesh(core_axis_name="c", subcore_axis_name="s")

    @functools.partial(
        pl.kernel,
        mesh=mesh,
        compiler_params=pltpu.CompilerParams(needs_layout_passes=False),
        out_type=jax.ShapeDtypeStruct((B,), jnp.float32),
        scratch_types=[
            pltpu.SMEM((b_per_w,), jnp.int32),
            pltpu.SMEM((b_per_w,), jnp.int32),
            pltpu.VMEM((b_per_w,), jnp.int32),
            pltpu.VMEM((b_per_w,), jnp.int32),
            pltpu.VMEM((E, D, 128), jnp.float32),
            pltpu.VMEM((E, D, 128), jnp.float32),
            pltpu.VMEM((E, D, 128), jnp.float32),
            pltpu.VMEM((E, D, 128), jnp.float32),
            pltpu.VMEM((b_per_w,), jnp.float32),
            pltpu.SemaphoreType.DMA,
            pltpu.SemaphoreType.DMA,
        ],
    )
    def sc_kernel(pid_hbm, sid_hbm, ptab_hbm, stab_hbm, out_hbm,
                  pidx_s, sidx_s, pidx_v, sidx_v,
                  pbufA, sbufA, pbufB, sbufB, outb_v, semA, semB):
        wid = lax.axis_index("s") * NC + lax.axis_index("c")
        base = wid * b_per_w
        pltpu.sync_copy(pid_hbm.at[pl.ds(base, b_per_w)], pidx_v)
        pltpu.sync_copy(sid_hbm.at[pl.ds(base, b_per_w)], sidx_v)

        def stage_body(g, carry):
            pv = pidx_v[pl.ds(g * L, L)]
            sv = sidx_v[pl.ds(g * L, L)]
            for j in range(L):
                pidx_s[g * L + j] = pv[j]
                sidx_s[g * L + j] = sv[j]
            return carry

        lax.fori_loop(0, b_per_w // L, stage_body, 0)

        iota = lax.iota(jnp.int32, L)
        ex16 = iota & (E - 1)
        dimq = (iota >> 2) * (D // 4)
        fold8 = jnp.where(iota < 8, iota + 8, iota)
        fold4 = jnp.where(iota < 4, iota + 4, iota)

        def fire(j, pbuf, sbuf, sem):
            @pl.when(j < n_sub)
            def _():
                for e in range(E):
                    pb = pidx_s[j * E + e] >> 7
                    pltpu.async_copy(
                        ptab_hbm.at[:, pl.ds(pl.multiple_of(pb * 128, 128), 128)],
                        pbuf.at[e], sem,
                    )
                    sb = sidx_s[j * E + e] >> 7
                    pltpu.async_copy(
                        stab_hbm.at[:, pl.ds(pl.multiple_of(sb * 128, 128), 128)],
                        sbuf.at[e], sem,
                    )

        def extract(j, pbuf, sbuf, sem):
            for e in range(E):
                pltpu.make_async_copy(
                    ptab_hbm.at[:, pl.ds(0, 128)], pbuf.at[e], sem
                ).wait()
                pltpu.make_async_copy(
                    stab_hbm.at[:, pl.ds(0, 128)], sbuf.at[e], sem
                ).wait()
            pl16 = pidx_v[pl.ds((j >> 2) * L, L)].at[
                (j & 3) * E + ex16
            ].get(mode="promise_in_bounds") & 127
            sl16 = sidx_v[pl.ds((j >> 2) * L, L)].at[
                (j & 3) * E + ex16
            ].get(mode="promise_in_bounds") & 127
            acc = jnp.zeros((L,), jnp.float32)
            for d in range(D // 4):
                dims = dimq + d
                pv = plsc.load_gather(pbuf, [ex16, dims, pl16])
                sv = plsc.load_gather(sbuf, [ex16, dims, sl16])
                acc = acc + pv * sv
            f1 = acc + acc.at[fold8].get(mode="promise_in_bounds")
            f2 = f1 + f1.at[fold4].get(mode="promise_in_bounds")
            return f2

        fire(0, pbufA, sbufA, semA)

        def pair_body(t, carry):
            j0 = t * 2
            fire(j0 + 1, pbufB, sbufB, semB)
            fa = extract(j0, pbufA, sbufA, semA)
            fire(j0 + 2, pbufA, sbufA, semA)
            fb = extract(j0 + 1, pbufB, sbufB, semB)
            pa = fa.at[iota & 3].get(mode="promise_in_bounds")
            pb = fb.at[iota & 3].get(mode="promise_in_bounds")
            qsel = iota >> 2
            merged = jnp.where(qsel == (j0 & 3), pa, carry)
            merged = jnp.where(qsel == ((j0 + 1) & 3), pb, merged)
            outb_v[pl.ds((t >> 1) * L, L)] = merged
            return merged

        lax.fori_loop(0, n_sub // 2, pair_body, jnp.zeros((L,), jnp.float32))
        pltpu.sync_copy(outb_v, out_hbm.at[pl.ds(base, b_per_w)])

    return sc_kernel(
        playlist_ids, song_ids, playlist_table.T, song_table.T
    )
